# baseline (speedup 1.0000x reference)
import jax
import jax.numpy as jnp
from jax import lax
from jax.experimental import pallas as pl
from jax.experimental.pallas import tpu as pltpu
from jax.experimental.pallas import tpu_sc as plsc

ROWS = 8192
COLS = 2048
R_TC = 5120
R_SC = ROWS - R_TC
BR = 1024
NB = R_TC // BR
NW = 32
RPW = R_SC // NW
CR = 16
NCH = RPW // CR


def _tc_body(pos_ref, img_ref, out_ref):
    pos = pos_ref[0, 0, :]
    cols = lax.broadcasted_iota(jnp.int32, (BR, COLS), 1)
    out_ref[:, :] = jnp.where(cols < pos[:, None], img_ref[:, :], 0.0)


def _sc_body(img_hbm, pos_hbm, out_hbm, buf, sem0, sem1, osem0, osem1):
    wid = lax.axis_index("s") * 2 + lax.axis_index("c")
    wrow = R_TC + wid * RPW
    orow = wid * RPW

    def start_in(c, b, sem):
        pltpu.make_async_copy(
            img_hbm.at[pl.ds(wrow + c * CR, CR)], buf.at[b], sem).start()

    def wait_in(b, sem):
        pltpu.make_async_copy(
            img_hbm.at[pl.ds(wrow, CR)], buf.at[b], sem).wait()

    def start_out(c, b, sem):
        pltpu.make_async_copy(
            buf.at[b], out_hbm.at[pl.ds(orow + c * CR, CR)], sem).start()

    def wait_out(b, sem):
        pltpu.make_async_copy(
            buf.at[b], out_hbm.at[pl.ds(orow, CR)], sem).wait()

    start_in(0, 0, sem0)
    start_in(1, 1, sem1)

    def step(c, b, sem, osem):
        wait_in(b, sem)
        start_out(c, b, osem)

        @pl.when(c + 2 < NCH)
        def _():
            wait_out(b, osem)
            start_in(c + 2, b, sem)

    def body(c, acc):
        @pl.when(c % 2 == 0)
        def _():
            step(c, 0, sem0, osem0)

        @pl.when(c % 2 == 1)
        def _():
            step(c, 1, sem1, osem1)

        return acc

    lax.fori_loop(0, NCH, body, 0)
    wait_out(0, osem0)
    wait_out(1, osem1)


@jax.jit
def _probe(images, position):
    mesh = plsc.VectorSubcoreMesh(core_axis_name="c", subcore_axis_name="s")
    sc = pl.kernel(
        _sc_body,
        out_type=jax.ShapeDtypeStruct((R_SC, COLS), jnp.float32),
        mesh=mesh,
        compiler_params=pltpu.CompilerParams(needs_layout_passes=False),
        scratch_types=[
            pltpu.VMEM((2, CR, COLS), jnp.float32),
            pltpu.SemaphoreType.DMA,
            pltpu.SemaphoreType.DMA,
            pltpu.SemaphoreType.DMA,
            pltpu.SemaphoreType.DMA,
        ],
    )
    sc_out = sc(images, position)

    pos3 = position[:R_TC].reshape(NB, 1, BR)
    tc_out = pl.pallas_call(
        _tc_body,
        grid=(NB,),
        in_specs=[
            pl.BlockSpec((1, 1, BR), lambda i: (i, 0, 0)),
            pl.BlockSpec((BR, COLS), lambda i: (i, 0)),
        ],
        out_specs=pl.BlockSpec((BR, COLS), lambda i: (i, 0)),
        out_shape=jax.ShapeDtypeStruct((R_TC, COLS), jnp.float32),
        compiler_params=pltpu.CompilerParams(
            dimension_semantics=("arbitrary",),
        ),
    )(pos3, images)
    return tc_out, sc_out


def kernel(images, position):
    return _probe(images, position)


# speedup vs baseline: 1.4405x; 1.4405x over previous
import jax
import jax.numpy as jnp
from jax import lax
from jax.experimental import pallas as pl
from jax.experimental.pallas import tpu as pltpu

ROWS = 8192
COLS = 2048
BR = 1024
NB = ROWS // BR


def _body(pos_ref, img_ref, out_ref):
    pos = pos_ref[0, 0, :]
    cols = lax.broadcasted_iota(jnp.int32, (BR, COLS), 1)
    out_ref[:, :] = jnp.where(cols < pos[:, None], img_ref[:, :], 0.0)


@jax.jit
def _call(images, position):
    pos3 = position.reshape(NB, 1, BR)
    return pl.pallas_call(
        _body,
        grid=(NB,),
        in_specs=[
            pl.BlockSpec((1, 1, BR), lambda i: (i, 0, 0)),
            pl.BlockSpec((BR, COLS), lambda i: (i, 0)),
        ],
        out_specs=pl.BlockSpec((BR, COLS), lambda i: (i, 0)),
        out_shape=jax.ShapeDtypeStruct((ROWS, COLS), jnp.float32),
        compiler_params=pltpu.CompilerParams(
            dimension_semantics=("arbitrary",),
        ),
    )(pos3, images)


def kernel(images, position):
    return _call(images, position)
